# baseline (speedup 1.0000x reference)
import functools

import jax
import jax.numpy as jnp
from jax import lax
from jax.experimental import pallas as pl
from jax.experimental.pallas import tpu as pltpu
from jax.experimental.pallas import tpu_sc as plsc

B = 4096
L = 50
D = 64
H = 256
C = 64

NC = 2
NS = 16
NW = NC * NS
BPW = B // NW
CB = 8
NCH = BPW // CB
RPC = CB * L
VPR = D // 16


def _sc_pool(ids_w, ids_2, ids_3, emb_w, emb_2, emb_3):
    mesh = plsc.VectorSubcoreMesh(core_axis_name="c", subcore_axis_name="s")
    out_t = tuple(jax.ShapeDtypeStruct((B, D), jnp.float32) for _ in range(3))
    NBUF = 4
    scratch = [
        pltpu.VMEM((BPW, L), jnp.int32),
        [pltpu.VMEM((RPC, D), jnp.float32) for _ in range(NBUF)],
        pltpu.VMEM((BPW, D), jnp.float32),
        [pltpu.SemaphoreType.DMA for _ in range(NBUF)],
    ]

    @functools.partial(pl.kernel, mesh=mesh, out_type=out_t,
                       scratch_types=scratch,
                       compiler_params=pltpu.CompilerParams(
                           use_tc_tiling_on_sc=False))
    def body(iw, i2, i3, tw, t2, t3, ow, o2, o3,
             idx_v, rows, acc_v, sems):
        cid = lax.axis_index("c")
        sid = lax.axis_index("s")
        wid = sid * NC + cid
        base = wid * BPW

        def accum(c, rbuf):
            row0 = c * CB

            def piece(k, j):
                return rbuf[(k // VPR) * L + j, pl.ds((k % VPR) * 16, 16)]

            def jbody(j, carry):
                return tuple(carry[k] + piece(k, j)
                             for k in range(CB * VPR))

            init = tuple(piece(k, 0) for k in range(CB * VPR))
            acc = lax.fori_loop(1, L, jbody, init, unroll=2)
            for k in range(CB * VPR):
                acc_v[row0 + k // VPR, pl.ds((k % VPR) * 16, 16)] = acc[k]

        for t, (ids, tab, out) in enumerate(
                ((iw, tw, ow), (i2, t2, o2), (i3, t3, o3))):
            pltpu.sync_copy(ids.at[pl.ds(base, BPW), :], idx_v)

            def subcopy(c, b, rbuf, tab=tab):
                return (tab.at[idx_v.at[c * CB + b, :]],
                        rbuf.at[pl.ds(b * L, L), :])

            def fire(c, rbuf, sem, subcopy=subcopy):
                for b in range(CB):
                    src, dst = subcopy(c, b, rbuf)
                    pltpu.make_async_copy(src, dst, sem).start()

            def drain(c, rbuf, sem, subcopy=subcopy):
                for b in range(CB):
                    src, dst = subcopy(c, b, rbuf)
                    pltpu.make_async_copy(src, dst, sem).wait()

            for d in range(NBUF):
                fire(d, rows[d], sems[d])

            def cbody(k, carry, fire=fire, drain=drain):
                for d in range(NBUF):
                    c = NBUF * k + d
                    drain(c, rows[d], sems[d])
                    accum(c, rows[d])

                    @pl.when(c + NBUF < NCH)
                    def _():
                        fire(c + NBUF, rows[d], sems[d])
                return carry

            lax.fori_loop(0, NCH // NBUF, cbody, 0)
            pltpu.sync_copy(acc_v, out.at[pl.ds(base, BPW)])

    return body(ids_w, ids_2, ids_3, emb_w, emb_2, emb_3)


BB = 256
NB = B // BB


def _mlp_kernel(pw, p2, p3, w1, b1, w2, b2, tgt, logits_ref, loss_ref):
    i = pl.program_id(0)
    x = jnp.concatenate([pw[...], p2[...], p3[...]], axis=-1) * (1.0 / L)
    h = jnp.dot(x, w1[...], preferred_element_type=jnp.float32,
                precision=lax.Precision.HIGHEST)
    h = jnp.maximum(h + b1[...], 0.0)
    logits = jnp.dot(h, w2[...], preferred_element_type=jnp.float32,
                     precision=lax.Precision.HIGHEST) + b2[...]
    logits_ref[...] = logits
    m = jnp.max(logits, axis=-1, keepdims=True)
    lse = jnp.log(jnp.sum(jnp.exp(logits - m), axis=-1, keepdims=True)) + m
    tgt_v = tgt[0, 0, :]
    onehot = lax.broadcasted_iota(jnp.int32, (BB, C), 1) == tgt_v[:, None]
    logit_t = jnp.sum(jnp.where(onehot, logits, 0.0), axis=-1, keepdims=True)
    nll_sum = jnp.sum(lse - logit_t, axis=(0, 1), keepdims=True)

    @pl.when(i == 0)
    def _():
        loss_ref[...] = jnp.zeros((1, 1), jnp.float32)

    loss_ref[...] += nll_sum * (1.0 / B)


def _mlp(pw, p2, p3, W1, b1, W2, b2, tgt):
    return pl.pallas_call(
        _mlp_kernel,
        grid=(NB,),
        in_specs=[
            pl.BlockSpec((BB, D), lambda i: (i, 0)),
            pl.BlockSpec((BB, D), lambda i: (i, 0)),
            pl.BlockSpec((BB, D), lambda i: (i, 0)),
            pl.BlockSpec((3 * D, H), lambda i: (0, 0)),
            pl.BlockSpec((1, H), lambda i: (0, 0)),
            pl.BlockSpec((H, C), lambda i: (0, 0)),
            pl.BlockSpec((1, C), lambda i: (0, 0)),
            pl.BlockSpec((1, 1, BB), lambda i: (i, 0, 0)),
        ],
        out_specs=[
            pl.BlockSpec((BB, C), lambda i: (i, 0)),
            pl.BlockSpec((1, 1), lambda i: (0, 0)),
        ],
        out_shape=[
            jax.ShapeDtypeStruct((B, C), jnp.float32),
            jax.ShapeDtypeStruct((1, 1), jnp.float32),
        ],
    )(pw, p2, p3, W1, b1, W2, b2, tgt)


def kernel(input_ids, input_mask, gram2_ids, gram3_ids, target,
           emb_word, emb_g2, emb_g3, W1, b1, W2, b2):
    pw, p2, p3 = _sc_pool(input_ids, gram2_ids, gram3_ids,
                          emb_word, emb_g2, emb_g3)
    logits, loss = _mlp(pw, p2, p3, W1,
                        b1.reshape(1, H), W2, b2.reshape(1, C),
                        target.reshape(NB, 1, BB))
    return (loss[0, 0], logits)


# speedup vs baseline: 1.0049x; 1.0049x over previous
import functools

import jax
import jax.numpy as jnp
from jax import lax
from jax.experimental import pallas as pl
from jax.experimental.pallas import tpu as pltpu
from jax.experimental.pallas import tpu_sc as plsc

B = 4096
L = 50
D = 64
H = 256
C = 64

NC = 2
NS = 16
NW = NC * NS
BPW = B // NW
CB = 8
NCH = BPW // CB
RPC = CB * L
VPR = D // 16


def _sc_pool(ids_w, ids_2, ids_3, emb_w, emb_2, emb_3):
    mesh = plsc.VectorSubcoreMesh(core_axis_name="c", subcore_axis_name="s")
    out_t = jax.ShapeDtypeStruct((B, 4 * D), jnp.float32)
    NBUF = 4
    scratch = [
        pltpu.VMEM((BPW, L), jnp.int32),
        [pltpu.VMEM((RPC, D), jnp.float32) for _ in range(NBUF)],
        pltpu.VMEM((BPW, D), jnp.float32),
        [pltpu.SemaphoreType.DMA for _ in range(NBUF)],
    ]

    @functools.partial(pl.kernel, mesh=mesh, out_type=out_t,
                       scratch_types=scratch,
                       compiler_params=pltpu.CompilerParams(
                           use_tc_tiling_on_sc=False))
    def body(iw, i2, i3, tw, t2, t3, out,
             idx_v, rows, acc_v, sems):
        cid = lax.axis_index("c")
        sid = lax.axis_index("s")
        wid = sid * NC + cid
        base = wid * BPW

        def accum(c, rbuf):
            row0 = c * CB

            def piece(k, j):
                return rbuf[(k // VPR) * L + j, pl.ds((k % VPR) * 16, 16)]

            def jbody(j, carry):
                return tuple(carry[k] + piece(k, j)
                             for k in range(CB * VPR))

            init = tuple(piece(k, 0) for k in range(CB * VPR))
            acc = lax.fori_loop(1, L, jbody, init, unroll=2)
            for k in range(CB * VPR):
                acc_v[row0 + k // VPR, pl.ds((k % VPR) * 16, 16)] = acc[k]

        for t, (ids, tab) in enumerate(((iw, tw), (i2, t2), (i3, t3))):
            pltpu.sync_copy(ids.at[pl.ds(base, BPW), :], idx_v)

            def subcopy(c, b, rbuf, tab=tab):
                return (tab.at[idx_v.at[c * CB + b, :]],
                        rbuf.at[pl.ds(b * L, L), :])

            def fire(c, rbuf, sem, subcopy=subcopy):
                for b in range(CB):
                    src, dst = subcopy(c, b, rbuf)
                    pltpu.make_async_copy(src, dst, sem).start()

            def drain(c, rbuf, sem, subcopy=subcopy):
                for b in range(CB):
                    src, dst = subcopy(c, b, rbuf)
                    pltpu.make_async_copy(src, dst, sem).wait()

            for d in range(NBUF):
                fire(d, rows[d], sems[d])

            def cbody(k, carry, fire=fire, drain=drain):
                for d in range(NBUF):
                    c = NBUF * k + d
                    drain(c, rows[d], sems[d])
                    accum(c, rows[d])

                    @pl.when(c + NBUF < NCH)
                    def _():
                        fire(c + NBUF, rows[d], sems[d])
                return carry

            lax.fori_loop(0, NCH // NBUF, cbody, 0)
            pltpu.sync_copy(acc_v,
                            out.at[pl.ds(base, BPW), pl.ds(t * D, D)])

    return body(ids_w, ids_2, ids_3, emb_w, emb_2, emb_3)


BB = 256
NB = B // BB


def _mlp_kernel(pooled, w1, b1, w2, b2, tgt, logits_ref, loss_ref):
    i = pl.program_id(0)
    x = pooled[:, :3 * D] * (1.0 / L)
    h = jnp.dot(x, w1[...], preferred_element_type=jnp.float32,
                precision=lax.Precision.HIGHEST)
    h = jnp.maximum(h + b1[...], 0.0)
    logits = jnp.dot(h, w2[...], preferred_element_type=jnp.float32,
                     precision=lax.Precision.HIGHEST) + b2[...]
    logits_ref[...] = logits
    m = jnp.max(logits, axis=-1, keepdims=True)
    lse = jnp.log(jnp.sum(jnp.exp(logits - m), axis=-1, keepdims=True)) + m
    tgt_v = tgt[0, 0, :]
    onehot = lax.broadcasted_iota(jnp.int32, (BB, C), 1) == tgt_v[:, None]
    logit_t = jnp.sum(jnp.where(onehot, logits, 0.0), axis=-1, keepdims=True)
    nll_sum = jnp.sum(lse - logit_t, axis=(0, 1), keepdims=True)

    @pl.when(i == 0)
    def _():
        loss_ref[...] = jnp.zeros((1, 1), jnp.float32)

    loss_ref[...] += nll_sum * (1.0 / B)


def _mlp(pooled, W1, b1, W2, b2, tgt):
    return pl.pallas_call(
        _mlp_kernel,
        grid=(NB,),
        in_specs=[
            pl.BlockSpec((BB, 4 * D), lambda i: (i, 0)),
            pl.BlockSpec((3 * D, H), lambda i: (0, 0)),
            pl.BlockSpec((1, H), lambda i: (0, 0)),
            pl.BlockSpec((H, C), lambda i: (0, 0)),
            pl.BlockSpec((1, C), lambda i: (0, 0)),
            pl.BlockSpec((1, 1, BB), lambda i: (i, 0, 0)),
        ],
        out_specs=[
            pl.BlockSpec((BB, C), lambda i: (i, 0)),
            pl.BlockSpec((1, 1), lambda i: (0, 0)),
        ],
        out_shape=[
            jax.ShapeDtypeStruct((B, C), jnp.float32),
            jax.ShapeDtypeStruct((1, 1), jnp.float32),
        ],
    )(pooled, W1, b1, W2, b2, tgt)


def kernel(input_ids, input_mask, gram2_ids, gram3_ids, target,
           emb_word, emb_g2, emb_g3, W1, b1, W2, b2):
    pooled = _sc_pool(input_ids, gram2_ids, gram3_ids,
                      emb_word, emb_g2, emb_g3)
    logits, loss = _mlp(pooled, W1,
                        b1.reshape(1, H), W2, b2.reshape(1, C),
                        target.reshape(NB, 1, BB))
    return (loss[0, 0], logits)
